# Initial kernel scaffold; baseline (speedup 1.0000x reference)
import jax
import jax.numpy as jnp
from jax.experimental import pallas as pl


def kernel(model_input, edge_index, W, a):
    raise NotImplementedError("write your pallas kernel here")


# speedup vs baseline: 8.1415x; 8.1415x over previous
import functools

import jax
import jax.numpy as jnp
from jax import lax
from jax.experimental import pallas as pl
from jax.experimental.pallas import tpu as pltpu
from jax.experimental.pallas import tpu_sc as plsc

N = 10000
E = 320000
F = 128
ALPHA = 0.2
C = 128
NCHUNKS = E // C
NW = 32
BASE_CHUNKS = NCHUNKS // NW
EXTRA = NCHUNKS - BASE_CHUNKS * NW
CP_ROWS = 1000


def _prep_body(x_ref, w_ref, a1_ref, a2_ref, h_ref, s1_ref, s2_ref):
    h = jnp.dot(x_ref[...], w_ref[...], preferred_element_type=jnp.float32)
    h_ref[...] = h
    s1_ref[...] = jnp.dot(h, a1_ref[...], preferred_element_type=jnp.float32)
    s2_ref[...] = jnp.dot(h, a2_ref[...], preferred_element_type=jnp.float32)


def _fin_body(acc_ref, rs_ref, o_ref):
    acc = acc_ref[0] + acc_ref[1]
    rs = rs_ref[0] + rs_ref[1]
    o_ref[...] = jnp.maximum(acc / rs, 0.0)


def _sc_body(h_hbm, s1_hbm, s2_hbm, ei_hbm, acc_out, rs_out,
             src_v, dst_v, s1v, s2v, eev, rows_v, zbuf, zbuf1,
             acc_sh, rs_sh, sem1, sem2, sem3):
    c = lax.axis_index("c")
    s = lax.axis_index("s")
    wid = s * 2 + c

    zv = jnp.zeros((16,), jnp.float32)

    def zrow(r, carry):
        for k in range(8):
            zbuf[r, pl.ds(k * 16, 16)] = zv
        return carry

    lax.fori_loop(0, 200, zrow, 0)

    def zrow1(i, carry):
        zbuf1[pl.ds(i * 16, 16)] = zv
        return carry

    lax.fori_loop(0, 63, zrow1, 0)

    @pl.when(s < 10)
    def _zero_spmem():
        for j in range(5):
            pltpu.sync_copy(zbuf, acc_sh.at[pl.ds(s * CP_ROWS + j * 200, 200)])
        pltpu.sync_copy(zbuf1.at[pl.ds(0, CP_ROWS)], rs_sh.at[pl.ds(s * CP_ROWS, CP_ROWS)])

    plsc.subcore_barrier()

    base = wid * BASE_CHUNKS + jnp.minimum(wid, EXTRA)
    count = jnp.where(wid < EXTRA, BASE_CHUNKS + 1, BASE_CHUNKS)

    def chunk_body(k, carry):
        e0 = (base + k) * C
        pltpu.sync_copy(ei_hbm.at[0, pl.ds(e0, C)], src_v)
        pltpu.sync_copy(ei_hbm.at[1, pl.ds(e0, C)], dst_v)
        g1 = pltpu.async_copy(s1_hbm.at[src_v], s1v, sem1)
        g2 = pltpu.async_copy(s2_hbm.at[dst_v], s2v, sem2)
        g3 = pltpu.async_copy(h_hbm.at[dst_v], rows_v, sem3)
        g1.wait()
        g2.wait()
        for g in range(C // 16):
            sl = pl.ds(g * 16, 16)
            x = s1v[sl] + s2v[sl]
            eev[sl] = jnp.exp(-jnp.maximum(x, ALPHA * x))
        g3.wait()

        def scale_body(e, carry2):
            eg = eev[pl.ds((e // 16) * 16, 16)]
            sv = lax.gather(
                eg, jnp.full((16, 1), e % 16, jnp.int32),
                lax.GatherDimensionNumbers(offset_dims=(),
                                           collapsed_slice_dims=(0,),
                                           start_index_map=(0,)),
                (1,), mode=lax.GatherScatterMode.PROMISE_IN_BOUNDS)
            for cc in range(F // 16):
                sl2 = pl.ds(cc * 16, 16)
                rows_v[e, sl2] = rows_v[e, sl2] * sv
            return carry2

        lax.fori_loop(0, C, scale_body, 0)
        pltpu.sync_copy(eev, rs_sh.at[src_v], add=True)
        pltpu.sync_copy(rows_v, acc_sh.at[src_v], add=True)
        return carry

    lax.fori_loop(0, count, chunk_body, 0)

    plsc.subcore_barrier()

    @pl.when(s < 10)
    def _copy_out():
        for j in range(5):
            pltpu.sync_copy(acc_sh.at[pl.ds(s * CP_ROWS + j * 200, 200)], zbuf)
            pltpu.sync_copy(zbuf, acc_out.at[c, pl.ds(s * CP_ROWS + j * 200, 200)])
        pltpu.sync_copy(rs_sh.at[pl.ds(s * CP_ROWS, CP_ROWS)],
                        zbuf1.at[pl.ds(0, CP_ROWS)])
        pltpu.sync_copy(zbuf1.at[pl.ds(0, CP_ROWS)],
                        rs_out.at[pl.ds(c * N + s * CP_ROWS, CP_ROWS)])


_sc_edges = pl.kernel(
    _sc_body,
    out_type=[
        jax.ShapeDtypeStruct((2, N, F), jnp.float32),
        jax.ShapeDtypeStruct((2 * N,), jnp.float32),
    ],
    mesh=plsc.VectorSubcoreMesh(core_axis_name="c", subcore_axis_name="s"),
    scratch_types=[
        pltpu.VMEM((C,), jnp.int32),
        pltpu.VMEM((C,), jnp.int32),
        pltpu.VMEM((C,), jnp.float32),
        pltpu.VMEM((C,), jnp.float32),
        pltpu.VMEM((C,), jnp.float32),
        pltpu.VMEM((C, F), jnp.float32),
        pltpu.VMEM((200, F), jnp.float32),
        pltpu.VMEM((1008,), jnp.float32),
        pltpu.VMEM_SHARED((N, F), jnp.float32),
        pltpu.VMEM_SHARED((N,), jnp.float32),
        pltpu.SemaphoreType.DMA,
        pltpu.SemaphoreType.DMA,
        pltpu.SemaphoreType.DMA,
    ],
)

_prep = pl.pallas_call(
    _prep_body,
    out_shape=[
        jax.ShapeDtypeStruct((N, F), jnp.float32),
        jax.ShapeDtypeStruct((N, 1), jnp.float32),
        jax.ShapeDtypeStruct((N, 1), jnp.float32),
    ],
)

_fin = pl.pallas_call(
    _fin_body,
    out_shape=jax.ShapeDtypeStruct((N, F), jnp.float32),
)


def kernel(model_input, edge_index, W, a):
    a1 = a[0, :F].reshape(F, 1)
    a2 = a[0, F:].reshape(F, 1)
    h, s1, s2 = _prep(model_input, W, a1, a2)
    acc, rs = _sc_edges(h, s1.reshape(N), s2.reshape(N), edge_index)
    return _fin(acc, rs.reshape(2, N, 1))


# speedup vs baseline: 9.2828x; 1.1402x over previous
import functools

import jax
import jax.numpy as jnp
from jax import lax
from jax.experimental import pallas as pl
from jax.experimental.pallas import tpu as pltpu
from jax.experimental.pallas import tpu_sc as plsc

N = 10000
E = 320000
F = 128
ALPHA = 0.2
C = 128
NCHUNKS = E // C
NW = 32
BASE_CHUNKS = NCHUNKS // NW
EXTRA = NCHUNKS - BASE_CHUNKS * NW
CP_ROWS = 1000


def _prep_body(x_ref, w_ref, a1_ref, a2_ref, h_ref, s1_ref, s2_ref):
    h = jnp.dot(x_ref[...], w_ref[...], preferred_element_type=jnp.float32)
    h_ref[...] = h
    s1_ref[...] = jnp.dot(h, a1_ref[...], preferred_element_type=jnp.float32)
    s2_ref[...] = jnp.dot(h, a2_ref[...], preferred_element_type=jnp.float32)


def _fin_body(acc_ref, rs_ref, o_ref):
    acc = acc_ref[0] + acc_ref[1]
    rs = rs_ref[0] + rs_ref[1]
    o_ref[...] = jnp.maximum(acc / rs, 0.0)


def _splat(vec, i):
    return lax.gather(
        vec, jnp.full((16, 1), i, jnp.int32),
        lax.GatherDimensionNumbers(offset_dims=(),
                                   collapsed_slice_dims=(0,),
                                   start_index_map=(0,)),
        (1,), mode=lax.GatherScatterMode.PROMISE_IN_BOUNDS)


def _sc_body(h_hbm, s1_hbm, s2_hbm, ei_hbm, acc_out, rs_out,
             src_v, dst_v, s1v, s2v, eev, rows_v, zbuf, zbuf1,
             acc_sh, rs_sh, sem1, sem2, sem3):
    c = lax.axis_index("c")
    s = lax.axis_index("s")
    wid = s * 2 + c

    zv = jnp.zeros((16,), jnp.float32)

    def zrow(r, carry):
        for k in range(8):
            zbuf[r, pl.ds(k * 16, 16)] = zv
        return carry

    lax.fori_loop(0, 200, zrow, 0)

    def zrow1(i, carry):
        zbuf1[pl.ds(i * 16, 16)] = zv
        return carry

    lax.fori_loop(0, 63, zrow1, 0)

    @pl.when(s < 10)
    def _zero_spmem():
        for j in range(5):
            pltpu.sync_copy(zbuf, acc_sh.at[pl.ds(s * CP_ROWS + j * 200, 200)])
        pltpu.sync_copy(zbuf1.at[pl.ds(0, CP_ROWS)], rs_sh.at[pl.ds(s * CP_ROWS, CP_ROWS)])

    plsc.subcore_barrier()

    base = wid * BASE_CHUNKS + jnp.minimum(wid, EXTRA)
    count = jnp.where(wid < EXTRA, BASE_CHUNKS + 1, BASE_CHUNKS)

    def chunk_body(k, carry):
        e0 = (base + k) * C
        pltpu.sync_copy(ei_hbm.at[0, pl.ds(e0, C)], src_v)
        pltpu.sync_copy(ei_hbm.at[1, pl.ds(e0, C)], dst_v)
        g1 = pltpu.async_copy(s1_hbm.at[src_v], s1v, sem1)
        g2 = pltpu.async_copy(s2_hbm.at[dst_v], s2v, sem2)
        g3 = pltpu.async_copy(h_hbm.at[dst_v], rows_v, sem3)
        g1.wait()
        g2.wait()

        def ee_body(g, carry2):
            sl = pl.ds(g * 16, 16)
            x = s1v[sl] + s2v[sl]
            eev[sl] = jnp.exp(-jnp.maximum(x, ALPHA * x))
            return carry2

        lax.fori_loop(0, C // 16, ee_body, 0)
        g3.wait()

        def scale_body(g, carry2):
            eg = eev[pl.ds(g * 16, 16)]
            for i in range(16):
                sv = _splat(eg, i)
                e = g * 16 + i
                for cc in range(F // 16):
                    sl2 = pl.ds(cc * 16, 16)
                    rows_v[e, sl2] = rows_v[e, sl2] * sv
            return carry2

        lax.fori_loop(0, C // 16, scale_body, 0)
        pltpu.sync_copy(eev, rs_sh.at[src_v], add=True)
        pltpu.sync_copy(rows_v, acc_sh.at[src_v], add=True)
        return carry

    lax.fori_loop(0, count, chunk_body, 0)

    plsc.subcore_barrier()

    @pl.when(s < 10)
    def _copy_out():
        for j in range(5):
            pltpu.sync_copy(acc_sh.at[pl.ds(s * CP_ROWS + j * 200, 200)], zbuf)
            pltpu.sync_copy(zbuf, acc_out.at[c, pl.ds(s * CP_ROWS + j * 200, 200)])
        pltpu.sync_copy(rs_sh.at[pl.ds(s * CP_ROWS, CP_ROWS)],
                        zbuf1.at[pl.ds(0, CP_ROWS)])
        pltpu.sync_copy(zbuf1.at[pl.ds(0, CP_ROWS)],
                        rs_out.at[pl.ds(c * N + s * CP_ROWS, CP_ROWS)])


_sc_edges = pl.kernel(
    _sc_body,
    out_type=[
        jax.ShapeDtypeStruct((2, N, F), jnp.float32),
        jax.ShapeDtypeStruct((2 * N,), jnp.float32),
    ],
    mesh=plsc.VectorSubcoreMesh(core_axis_name="c", subcore_axis_name="s"),
    scratch_types=[
        pltpu.VMEM((C,), jnp.int32),
        pltpu.VMEM((C,), jnp.int32),
        pltpu.VMEM((C,), jnp.float32),
        pltpu.VMEM((C,), jnp.float32),
        pltpu.VMEM((C,), jnp.float32),
        pltpu.VMEM((C, F), jnp.float32),
        pltpu.VMEM((200, F), jnp.float32),
        pltpu.VMEM((1008,), jnp.float32),
        pltpu.VMEM_SHARED((N, F), jnp.float32),
        pltpu.VMEM_SHARED((N,), jnp.float32),
        pltpu.SemaphoreType.DMA,
        pltpu.SemaphoreType.DMA,
        pltpu.SemaphoreType.DMA,
    ],
)

_prep = pl.pallas_call(
    _prep_body,
    out_shape=[
        jax.ShapeDtypeStruct((N, F), jnp.float32),
        jax.ShapeDtypeStruct((N, 1), jnp.float32),
        jax.ShapeDtypeStruct((N, 1), jnp.float32),
    ],
)

_fin = pl.pallas_call(
    _fin_body,
    out_shape=jax.ShapeDtypeStruct((N, F), jnp.float32),
)


def kernel(model_input, edge_index, W, a):
    a1 = a[0, :F].reshape(F, 1)
    a2 = a[0, F:].reshape(F, 1)
    h, s1, s2 = _prep(model_input, W, a1, a2)
    acc, rs = _sc_edges(h, s1.reshape(N), s2.reshape(N), edge_index)
    return _fin(acc, rs.reshape(2, N, 1))


# speedup vs baseline: 13.1958x; 1.4215x over previous
import functools

import jax
import jax.numpy as jnp
from jax import lax
from jax.experimental import pallas as pl
from jax.experimental.pallas import tpu as pltpu
from jax.experimental.pallas import tpu_sc as plsc

N = 10000
E = 320000
F = 128
ALPHA = 0.2
C = 128
NCHUNKS = E // C
NW = 32
BASE_CHUNKS = NCHUNKS // NW
EXTRA = NCHUNKS - BASE_CHUNKS * NW
CP_ROWS = 1000


def _prep_body(x_ref, w_ref, a1_ref, a2_ref, h_ref, s1_ref, s2_ref):
    h = jnp.dot(x_ref[...], w_ref[...], preferred_element_type=jnp.float32)
    h_ref[...] = h
    s1_ref[...] = jnp.dot(h, a1_ref[...], preferred_element_type=jnp.float32)
    s2_ref[...] = jnp.dot(h, a2_ref[...], preferred_element_type=jnp.float32)


def _fin_body(acc_ref, rs_ref, o_ref):
    acc = acc_ref[0] + acc_ref[1]
    rs = rs_ref[0] + rs_ref[1]
    o_ref[...] = jnp.maximum(acc / rs, 0.0)


def _splat(vec, i):
    return lax.gather(
        vec, jnp.full((16, 1), i, jnp.int32),
        lax.GatherDimensionNumbers(offset_dims=(),
                                   collapsed_slice_dims=(0,),
                                   start_index_map=(0,)),
        (1,), mode=lax.GatherScatterMode.PROMISE_IN_BOUNDS)


def _sc_body(h_hbm, s1_hbm, s2_hbm, ei_hbm, acc_out, rs_out,
             *bufs):
    (srcA, dstA, s1A, s2A, eeA, rowsA, g1A, g2A, g3A, r1A, r2A,
     srcB, dstB, s1B, s2B, eeB, rowsB, g1B, g2B, g3B, r1B, r2B,
     zbuf1, acc_sh, rs_sh) = bufs
    A = (srcA, dstA, s1A, s2A, eeA, rowsA, g1A, g2A, g3A, r1A, r2A)
    B = (srcB, dstB, s1B, s2B, eeB, rowsB, g1B, g2B, g3B, r1B, r2B)
    c = lax.axis_index("c")
    s = lax.axis_index("s")
    wid = s * 2 + c

    zv = jnp.zeros((16,), jnp.float32)

    def zrow(r, carry):
        for k in range(8):
            rowsA[r, pl.ds(k * 16, 16)] = zv
        return carry

    lax.fori_loop(0, C, zrow, 0)

    def zrow1(i, carry):
        zbuf1[pl.ds(i * 16, 16)] = zv
        return carry

    lax.fori_loop(0, 63, zrow1, 0)

    @pl.when(s < 10)
    def _zero_spmem():
        for j in range(7):
            pltpu.sync_copy(rowsA, acc_sh.at[pl.ds(s * CP_ROWS + j * C, C)])
        pltpu.sync_copy(rowsA.at[pl.ds(0, 104)],
                        acc_sh.at[pl.ds(s * CP_ROWS + 7 * C, 104)])
        pltpu.sync_copy(zbuf1.at[pl.ds(0, CP_ROWS)], rs_sh.at[pl.ds(s * CP_ROWS, CP_ROWS)])

    plsc.subcore_barrier()

    base = wid * BASE_CHUNKS

    def issue(cid, bs):
        src_v, dst_v, s1v, s2v, eev, rows_v, g1, g2, g3, r1, r2 = bs
        e0 = cid * C
        pltpu.sync_copy(ei_hbm.at[0, pl.ds(e0, C)], src_v)
        pltpu.sync_copy(ei_hbm.at[1, pl.ds(e0, C)], dst_v)
        pltpu.async_copy(s1_hbm.at[src_v], s1v, g1)
        pltpu.async_copy(s2_hbm.at[dst_v], s2v, g2)
        pltpu.async_copy(h_hbm.at[dst_v], rows_v, g3)

    def process(bs):
        src_v, dst_v, s1v, s2v, eev, rows_v, g1, g2, g3, r1, r2 = bs
        pltpu.make_async_copy(s1_hbm.at[src_v], s1v, g1).wait()
        pltpu.make_async_copy(s2_hbm.at[dst_v], s2v, g2).wait()

        def ee_body(g, carry2):
            sl = pl.ds(g * 16, 16)
            x = s1v[sl] + s2v[sl]
            eev[sl] = jnp.exp(-jnp.maximum(x, ALPHA * x))
            return carry2

        lax.fori_loop(0, C // 16, ee_body, 0)
        pltpu.make_async_copy(h_hbm.at[dst_v], rows_v, g3).wait()

        def scale_body(g, carry2):
            eg = eev[pl.ds(g * 16, 16)]
            for i in range(16):
                sv = _splat(eg, i)
                e = g * 16 + i
                for cc in range(F // 16):
                    sl2 = pl.ds(cc * 16, 16)
                    rows_v[e, sl2] = rows_v[e, sl2] * sv
            return carry2

        lax.fori_loop(0, C // 16, scale_body, 0)
        pltpu.async_copy(eev, rs_sh.at[src_v], r1, add=True)
        pltpu.async_copy(rows_v, acc_sh.at[src_v], r2, add=True)

    def wait_scatter(bs):
        src_v, dst_v, s1v, s2v, eev, rows_v, g1, g2, g3, r1, r2 = bs
        pltpu.make_async_copy(eev, rs_sh.at[src_v], r1).wait()
        pltpu.make_async_copy(rows_v, acc_sh.at[src_v], r2).wait()

    issue(base, A)

    def pair_body(k2, carry):
        c0 = base + 2 * k2
        issue(c0 + 1, B)
        process(A)
        wait_scatter(A)

        @pl.when(k2 < BASE_CHUNKS // 2 - 1)
        def _prefetch_a():
            issue(c0 + 2, A)

        process(B)
        wait_scatter(B)
        return carry

    lax.fori_loop(0, BASE_CHUNKS // 2, pair_body, 0)

    @pl.when(wid < EXTRA)
    def _tail():
        issue(NW * BASE_CHUNKS + wid, A)
        process(A)
        wait_scatter(A)

    plsc.subcore_barrier()

    @pl.when(s < 10)
    def _copy_out():
        for j in range(7):
            pltpu.sync_copy(acc_sh.at[pl.ds(s * CP_ROWS + j * C, C)], rowsA)
            pltpu.sync_copy(rowsA, acc_out.at[c, pl.ds(s * CP_ROWS + j * C, C)])
        pltpu.sync_copy(acc_sh.at[pl.ds(s * CP_ROWS + 7 * C, 104)],
                        rowsA.at[pl.ds(0, 104)])
        pltpu.sync_copy(rowsA.at[pl.ds(0, 104)],
                        acc_out.at[c, pl.ds(s * CP_ROWS + 7 * C, 104)])
        pltpu.sync_copy(rs_sh.at[pl.ds(s * CP_ROWS, CP_ROWS)],
                        zbuf1.at[pl.ds(0, CP_ROWS)])
        pltpu.sync_copy(zbuf1.at[pl.ds(0, CP_ROWS)],
                        rs_out.at[pl.ds(c * N + s * CP_ROWS, CP_ROWS)])


_sc_edges = pl.kernel(
    _sc_body,
    out_type=[
        jax.ShapeDtypeStruct((2, N, F), jnp.float32),
        jax.ShapeDtypeStruct((2 * N,), jnp.float32),
    ],
    mesh=plsc.VectorSubcoreMesh(core_axis_name="c", subcore_axis_name="s"),
    scratch_types=(
        2 * [
            pltpu.VMEM((C,), jnp.int32),
            pltpu.VMEM((C,), jnp.int32),
            pltpu.VMEM((C,), jnp.float32),
            pltpu.VMEM((C,), jnp.float32),
            pltpu.VMEM((C,), jnp.float32),
            pltpu.VMEM((C, F), jnp.float32),
            pltpu.SemaphoreType.DMA,
            pltpu.SemaphoreType.DMA,
            pltpu.SemaphoreType.DMA,
            pltpu.SemaphoreType.DMA,
            pltpu.SemaphoreType.DMA,
        ]
        + [
            pltpu.VMEM((1008,), jnp.float32),
            pltpu.VMEM_SHARED((N, F), jnp.float32),
            pltpu.VMEM_SHARED((N,), jnp.float32),
        ]
    ),
)

_prep = pl.pallas_call(
    _prep_body,
    out_shape=[
        jax.ShapeDtypeStruct((N, F), jnp.float32),
        jax.ShapeDtypeStruct((N, 1), jnp.float32),
        jax.ShapeDtypeStruct((N, 1), jnp.float32),
    ],
)

_fin = pl.pallas_call(
    _fin_body,
    out_shape=jax.ShapeDtypeStruct((N, F), jnp.float32),
)


def kernel(model_input, edge_index, W, a):
    a1 = a[0, :F].reshape(F, 1)
    a2 = a[0, F:].reshape(F, 1)
    h, s1, s2 = _prep(model_input, W, a1, a2)
    acc, rs = _sc_edges(h, s1.reshape(N), s2.reshape(N), edge_index)
    return _fin(acc, rs.reshape(2, N, 1))


# speedup vs baseline: 17.2876x; 1.3101x over previous
import jax
import jax.numpy as jnp
from jax import lax
from jax.experimental import pallas as pl
from jax.experimental.pallas import tpu as pltpu
from jax.experimental.pallas import tpu_sc as plsc

N = 10000
E = 320000
F = 128
ALPHA = 0.2
C = 128
NCHUNKS = E // C
NW = 32
BASE_CHUNKS = NCHUNKS // NW
EXTRA = NCHUNKS - BASE_CHUNKS * NW
CP_ROWS = 1000


def _prep_body(x_ref, w_ref, a1_ref, a2_ref, h_ref, s1_ref, s2_ref):
    h = jnp.dot(x_ref[...], w_ref[...], preferred_element_type=jnp.float32)
    h_ref[...] = h
    s1_ref[...] = jnp.dot(h, a1_ref[...], preferred_element_type=jnp.float32)
    s2_ref[...] = jnp.dot(h, a2_ref[...], preferred_element_type=jnp.float32)


def _fin_body(acc_ref, rs_ref, o_ref):
    acc = acc_ref[0] + acc_ref[1]
    rs = rs_ref[0] + rs_ref[1]
    o_ref[...] = jnp.maximum(acc / rs, 0.0)


def _splat(vec, i):
    return lax.gather(
        vec, jnp.full((16, 1), i, jnp.int32),
        lax.GatherDimensionNumbers(offset_dims=(),
                                   collapsed_slice_dims=(0,),
                                   start_index_map=(0,)),
        (1,), mode=lax.GatherScatterMode.PROMISE_IN_BOUNDS)


def _sc_body(h_hbm, s1_hbm, s2_hbm, ei_hbm, acc_out, rs_out, *bufs):
    (s1A, s2A, eeA, rowsA, g1A, g2A, g3A, r1A, r2A,
     s1B, s2B, eeB, rowsB, g1B, g2B, g3B, r1B, r2B,
     ix0, gi0, ix1, gi1, ix2, gi2,
     zbuf1, acc_sh, rs_sh) = bufs
    A = (s1A, s2A, eeA, rowsA, g1A, g2A, g3A, r1A, r2A)
    B = (s1B, s2B, eeB, rowsB, g1B, g2B, g3B, r1B, r2B)
    I = ((ix0, gi0), (ix1, gi1), (ix2, gi2))
    c = lax.axis_index("c")
    s = lax.axis_index("s")
    wid = s * 2 + c

    zv = jnp.zeros((16,), jnp.float32)

    def zrow(r, carry):
        for k in range(8):
            rowsA[r, pl.ds(k * 16, 16)] = zv
        return carry

    lax.fori_loop(0, C, zrow, 0)

    def zrow1(i, carry):
        zbuf1[pl.ds(i * 16, 16)] = zv
        return carry

    lax.fori_loop(0, 63, zrow1, 0)

    @pl.when(s < 10)
    def _zero_spmem():
        for j in range(7):
            pltpu.sync_copy(rowsA, acc_sh.at[pl.ds(s * CP_ROWS + j * C, C)])
        pltpu.sync_copy(rowsA.at[pl.ds(0, 104)],
                        acc_sh.at[pl.ds(s * CP_ROWS + 7 * C, 104)])
        pltpu.sync_copy(zbuf1.at[pl.ds(0, CP_ROWS)], rs_sh.at[pl.ds(s * CP_ROWS, CP_ROWS)])

    plsc.subcore_barrier()

    base = wid * BASE_CHUNKS

    def issue_idx(cid, iset):
        idx2, gi = iset
        pltpu.async_copy(ei_hbm.at[:, pl.ds(cid * C, C)], idx2, gi)

    def wait_idx(cid, iset):
        idx2, gi = iset
        pltpu.make_async_copy(ei_hbm.at[:, pl.ds(cid * C, C)], idx2, gi).wait()

    def issue_gathers(ds, iset):
        s1v, s2v, eev, rows_v, g1, g2, g3, r1, r2 = ds
        idx2, gi = iset
        pltpu.async_copy(s1_hbm.at[idx2.at[0]], s1v, g1)
        pltpu.async_copy(s2_hbm.at[idx2.at[1]], s2v, g2)
        pltpu.async_copy(h_hbm.at[idx2.at[1]], rows_v, g3)

    def process(ds, iset):
        s1v, s2v, eev, rows_v, g1, g2, g3, r1, r2 = ds
        idx2, gi = iset
        pltpu.make_async_copy(s1_hbm.at[idx2.at[0]], s1v, g1).wait()
        pltpu.make_async_copy(s2_hbm.at[idx2.at[1]], s2v, g2).wait()
        for g in range(C // 16):
            sl = pl.ds(g * 16, 16)
            x = s1v[sl] + s2v[sl]
            eev[sl] = jnp.exp(-jnp.maximum(x, ALPHA * x))
        pltpu.make_async_copy(h_hbm.at[idx2.at[1]], rows_v, g3).wait()

        def scale_body(g, carry2):
            eg = eev[pl.ds(g * 16, 16)]
            for i in range(16):
                sv = _splat(eg, i)
                e = g * 16 + i
                for cc in range(F // 16):
                    sl2 = pl.ds(cc * 16, 16)
                    rows_v[e, sl2] = rows_v[e, sl2] * sv
            return carry2

        lax.fori_loop(0, C // 16, scale_body, 0)
        pltpu.async_copy(eev, rs_sh.at[idx2.at[0]], r1, add=True)
        pltpu.async_copy(rows_v, acc_sh.at[idx2.at[0]], r2, add=True)

    def wait_scatter(ds, iset):
        s1v, s2v, eev, rows_v, g1, g2, g3, r1, r2 = ds
        idx2, gi = iset
        pltpu.make_async_copy(eev, rs_sh.at[idx2.at[0]], r1).wait()
        pltpu.make_async_copy(rows_v, acc_sh.at[idx2.at[0]], r2).wait()

    issue_idx(base, I[0])
    issue_idx(base + 1, I[1])
    wait_idx(base, I[0])
    issue_gathers(A, I[0])

    def six_body(k, carry):
        j0 = base + 6 * k
        for jj in range(6):
            d_cur = (A, B)[jj % 2]
            d_nxt = (A, B)[(jj + 1) % 2]
            i_cur = I[jj % 3]
            i_nxt = I[(jj + 1) % 3]
            i_nx2 = I[(jj + 2) % 3]

            if jj > 0:
                wait_scatter(d_nxt, i_nx2)
            else:
                @pl.when(k > 0)
                def _drain_prev():
                    wait_scatter(d_nxt, i_nx2)

            wait_idx(j0 + jj + 1, i_nxt)
            issue_gathers(d_nxt, i_nxt)
            issue_idx(j0 + jj + 2, i_nx2)
            process(d_cur, i_cur)
        return carry

    lax.fori_loop(0, BASE_CHUNKS // 6, six_body, 0)

    wait_scatter(B, I[2])
    pltpu.make_async_copy(s1_hbm.at[ix0.at[0]], s1A, g1A).wait()
    pltpu.make_async_copy(s2_hbm.at[ix0.at[1]], s2A, g2A).wait()
    pltpu.make_async_copy(h_hbm.at[ix0.at[1]], rowsA, g3A).wait()
    wait_idx(base + BASE_CHUNKS + 1, I[1])

    @pl.when(wid < EXTRA)
    def _tail():
        cid = NW * BASE_CHUNKS + wid
        issue_idx(cid, I[0])
        wait_idx(cid, I[0])
        issue_gathers(A, I[0])
        process(A, I[0])
        wait_scatter(A, I[0])

    plsc.subcore_barrier()

    @pl.when(s < 10)
    def _copy_out():
        for j in range(7):
            pltpu.sync_copy(acc_sh.at[pl.ds(s * CP_ROWS + j * C, C)], rowsA)
            pltpu.sync_copy(rowsA, acc_out.at[c, pl.ds(s * CP_ROWS + j * C, C)])
        pltpu.sync_copy(acc_sh.at[pl.ds(s * CP_ROWS + 7 * C, 104)],
                        rowsA.at[pl.ds(0, 104)])
        pltpu.sync_copy(rowsA.at[pl.ds(0, 104)],
                        acc_out.at[c, pl.ds(s * CP_ROWS + 7 * C, 104)])
        pltpu.sync_copy(rs_sh.at[pl.ds(s * CP_ROWS, CP_ROWS)],
                        zbuf1.at[pl.ds(0, CP_ROWS)])
        pltpu.sync_copy(zbuf1.at[pl.ds(0, CP_ROWS)],
                        rs_out.at[pl.ds(c * N + s * CP_ROWS, CP_ROWS)])


_data_set = [
    pltpu.VMEM((C,), jnp.float32),
    pltpu.VMEM((C,), jnp.float32),
    pltpu.VMEM((C,), jnp.float32),
    pltpu.VMEM((C, F), jnp.float32),
    pltpu.SemaphoreType.DMA,
    pltpu.SemaphoreType.DMA,
    pltpu.SemaphoreType.DMA,
    pltpu.SemaphoreType.DMA,
    pltpu.SemaphoreType.DMA,
]
_idx_set = [
    pltpu.VMEM((2, C), jnp.int32),
    pltpu.SemaphoreType.DMA,
]

_sc_edges = pl.kernel(
    _sc_body,
    out_type=[
        jax.ShapeDtypeStruct((2, N, F), jnp.float32),
        jax.ShapeDtypeStruct((2 * N,), jnp.float32),
    ],
    mesh=plsc.VectorSubcoreMesh(core_axis_name="c", subcore_axis_name="s"),
    scratch_types=(
        2 * _data_set
        + 3 * _idx_set
        + [
            pltpu.VMEM((1008,), jnp.float32),
            pltpu.VMEM_SHARED((N, F), jnp.float32),
            pltpu.VMEM_SHARED((N,), jnp.float32),
        ]
    ),
)

_prep = pl.pallas_call(
    _prep_body,
    out_shape=[
        jax.ShapeDtypeStruct((N, F), jnp.float32),
        jax.ShapeDtypeStruct((N, 1), jnp.float32),
        jax.ShapeDtypeStruct((N, 1), jnp.float32),
    ],
)

_fin = pl.pallas_call(
    _fin_body,
    out_shape=jax.ShapeDtypeStruct((N, F), jnp.float32),
)


def kernel(model_input, edge_index, W, a):
    a1 = a[0, :F].reshape(F, 1)
    a2 = a[0, F:].reshape(F, 1)
    h, s1, s2 = _prep(model_input, W, a1, a2)
    acc, rs = _sc_edges(h, s1.reshape(N), s2.reshape(N), edge_index)
    return _fin(acc, rs.reshape(2, N, 1))


# speedup vs baseline: 17.9547x; 1.0386x over previous
import jax
import jax.numpy as jnp
from jax import lax
from jax.experimental import pallas as pl
from jax.experimental.pallas import tpu as pltpu
from jax.experimental.pallas import tpu_sc as plsc

N = 10000
E = 320000
F = 128
ALPHA = 0.2
C = 128
NCHUNKS = E // C
NW = 32
BASE_CHUNKS = NCHUNKS // NW
EXTRA = NCHUNKS - BASE_CHUNKS * NW
CP_ROWS = 1000
NPAD = 10240


def _prep_body(x_ref, w_ref, a1_ref, a2_ref, h_ref, s1_ref, s2_ref):
    h = jnp.dot(x_ref[...], w_ref[...], preferred_element_type=jnp.float32)
    h_ref[...] = h
    s1_ref[...] = jnp.dot(h, a1_ref[...], preferred_element_type=jnp.float32)
    s2_ref[...] = jnp.dot(h, a2_ref[...], preferred_element_type=jnp.float32)


def _fin_body(acc_ref, rs_ref, o_ref):
    acc = acc_ref[0][:N] + acc_ref[1][:N]
    rs = rs_ref[0] + rs_ref[1]
    o_ref[...] = jnp.maximum(acc / rs, 0.0)


def _splat(vec, i):
    return lax.gather(
        vec, jnp.full((16, 1), i, jnp.int32),
        lax.GatherDimensionNumbers(offset_dims=(),
                                   collapsed_slice_dims=(0,),
                                   start_index_map=(0,)),
        (1,), mode=lax.GatherScatterMode.PROMISE_IN_BOUNDS)


def _sc_body(h_hbm, s1_hbm, s2_hbm, ei_hbm, acc_out, rs_out, *bufs):
    (s1A, s2A, eeA, rowsA, g1A, g2A, g3A, r1A, r2A,
     s1B, s2B, eeB, rowsB, g1B, g2B, g3B, r1B, r2B,
     ix0, gi0, ix1, gi1, ix2, gi2,
     zbuf1, acc_sh, rs_sh) = bufs
    A = (s1A, s2A, eeA, rowsA, g1A, g2A, g3A, r1A, r2A)
    B = (s1B, s2B, eeB, rowsB, g1B, g2B, g3B, r1B, r2B)
    I = ((ix0, gi0), (ix1, gi1), (ix2, gi2))
    c = lax.axis_index("c")
    s = lax.axis_index("s")
    wid = s * 2 + c

    zv = jnp.zeros((16,), jnp.float32)

    def zrow(r, carry):
        for k in range(8):
            rowsA[r, pl.ds(k * 16, 16)] = zv
        return carry

    lax.fori_loop(0, C, zrow, 0)

    def zrow1(i, carry):
        zbuf1[pl.ds(i * 16, 16)] = zv
        return carry

    lax.fori_loop(0, 63, zrow1, 0)

    for j in range(5):
        pltpu.async_copy(rowsA, acc_sh.at[pl.ds(s * 640 + j * C, C)], g1A)
    for j in range(5):
        pltpu.make_async_copy(rowsA, acc_sh.at[pl.ds(s * 640 + j * C, C)], g1A).wait()

    @pl.when(s < 10)
    def _zero_spmem():
        pltpu.sync_copy(zbuf1.at[pl.ds(0, CP_ROWS)], rs_sh.at[pl.ds(s * CP_ROWS, CP_ROWS)])

    plsc.subcore_barrier()

    base = wid * BASE_CHUNKS

    def issue_idx(cid, iset):
        idx2, gi = iset
        pltpu.async_copy(ei_hbm.at[:, pl.ds(cid * C, C)], idx2, gi)

    def wait_idx(cid, iset):
        idx2, gi = iset
        pltpu.make_async_copy(ei_hbm.at[:, pl.ds(cid * C, C)], idx2, gi).wait()

    def issue_gathers(ds, iset):
        s1v, s2v, eev, rows_v, g1, g2, g3, r1, r2 = ds
        idx2, gi = iset
        pltpu.async_copy(s1_hbm.at[idx2.at[0]], s1v, g1)
        pltpu.async_copy(s2_hbm.at[idx2.at[1]], s2v, g2)
        pltpu.async_copy(h_hbm.at[idx2.at[1]], rows_v, g3)

    def process(ds, iset):
        s1v, s2v, eev, rows_v, g1, g2, g3, r1, r2 = ds
        idx2, gi = iset
        pltpu.make_async_copy(s1_hbm.at[idx2.at[0]], s1v, g1).wait()
        pltpu.make_async_copy(s2_hbm.at[idx2.at[1]], s2v, g2).wait()
        for g in range(C // 16):
            sl = pl.ds(g * 16, 16)
            x = s1v[sl] + s2v[sl]
            eev[sl] = jnp.exp(-jnp.maximum(x, ALPHA * x))
        pltpu.make_async_copy(h_hbm.at[idx2.at[1]], rows_v, g3).wait()

        def scale_body(g, carry2):
            eg = eev[pl.ds(g * 16, 16)]
            for i in range(16):
                sv = _splat(eg, i)
                e = g * 16 + i
                for cc in range(F // 16):
                    sl2 = pl.ds(cc * 16, 16)
                    rows_v[e, sl2] = rows_v[e, sl2] * sv
            return carry2

        lax.fori_loop(0, C // 16, scale_body, 0)
        pltpu.async_copy(eev, rs_sh.at[idx2.at[0]], r1, add=True)
        pltpu.async_copy(rows_v, acc_sh.at[idx2.at[0]], r2, add=True)

    def wait_scatter(ds, iset):
        s1v, s2v, eev, rows_v, g1, g2, g3, r1, r2 = ds
        idx2, gi = iset
        pltpu.make_async_copy(eev, rs_sh.at[idx2.at[0]], r1).wait()
        pltpu.make_async_copy(rows_v, acc_sh.at[idx2.at[0]], r2).wait()

    issue_idx(base, I[0])
    issue_idx(base + 1, I[1])
    wait_idx(base, I[0])
    issue_gathers(A, I[0])

    def six_body(k, carry):
        j0 = base + 6 * k
        for jj in range(6):
            d_cur = (A, B)[jj % 2]
            d_nxt = (A, B)[(jj + 1) % 2]
            i_cur = I[jj % 3]
            i_nxt = I[(jj + 1) % 3]
            i_nx2 = I[(jj + 2) % 3]

            if jj > 0:
                wait_scatter(d_nxt, i_nx2)
            else:
                @pl.when(k > 0)
                def _drain_prev():
                    wait_scatter(d_nxt, i_nx2)

            wait_idx(j0 + jj + 1, i_nxt)
            issue_gathers(d_nxt, i_nxt)
            issue_idx(j0 + jj + 2, i_nx2)
            process(d_cur, i_cur)
        return carry

    lax.fori_loop(0, BASE_CHUNKS // 6, six_body, 0)

    wait_scatter(B, I[2])
    pltpu.make_async_copy(s1_hbm.at[ix0.at[0]], s1A, g1A).wait()
    pltpu.make_async_copy(s2_hbm.at[ix0.at[1]], s2A, g2A).wait()
    pltpu.make_async_copy(h_hbm.at[ix0.at[1]], rowsA, g3A).wait()
    wait_idx(base + BASE_CHUNKS + 1, I[1])

    @pl.when(wid < EXTRA)
    def _tail():
        cid = NW * BASE_CHUNKS + wid
        issue_idx(cid, I[0])
        wait_idx(cid, I[0])
        issue_gathers(A, I[0])
        process(A, I[0])
        wait_scatter(A, I[0])

    plsc.subcore_barrier()

    stages = ((rowsA, g1A), (rowsB, g1B))
    for j in range(5):
        stage, sem = stages[j % 2]
        if j >= 2:
            pltpu.make_async_copy(
                stage, acc_out.at[c, pl.ds(s * 640 + (j - 2) * C, C)], sem).wait()
        pltpu.sync_copy(acc_sh.at[pl.ds(s * 640 + j * C, C)], stage)
        pltpu.async_copy(stage, acc_out.at[c, pl.ds(s * 640 + j * C, C)], sem)
    for j in (3, 4):
        stage, sem = stages[j % 2]
        pltpu.make_async_copy(
            stage, acc_out.at[c, pl.ds(s * 640 + j * C, C)], sem).wait()

    @pl.when(s < 10)
    def _copy_out_rs():
        pltpu.sync_copy(rs_sh.at[pl.ds(s * CP_ROWS, CP_ROWS)],
                        zbuf1.at[pl.ds(0, CP_ROWS)])
        pltpu.sync_copy(zbuf1.at[pl.ds(0, CP_ROWS)],
                        rs_out.at[pl.ds(c * N + s * CP_ROWS, CP_ROWS)])


_data_set = [
    pltpu.VMEM((C,), jnp.float32),
    pltpu.VMEM((C,), jnp.float32),
    pltpu.VMEM((C,), jnp.float32),
    pltpu.VMEM((C, F), jnp.float32),
    pltpu.SemaphoreType.DMA,
    pltpu.SemaphoreType.DMA,
    pltpu.SemaphoreType.DMA,
    pltpu.SemaphoreType.DMA,
    pltpu.SemaphoreType.DMA,
]
_idx_set = [
    pltpu.VMEM((2, C), jnp.int32),
    pltpu.SemaphoreType.DMA,
]

_sc_edges = pl.kernel(
    _sc_body,
    out_type=[
        jax.ShapeDtypeStruct((2, NPAD, F), jnp.float32),
        jax.ShapeDtypeStruct((2 * N,), jnp.float32),
    ],
    mesh=plsc.VectorSubcoreMesh(core_axis_name="c", subcore_axis_name="s"),
    scratch_types=(
        2 * _data_set
        + 3 * _idx_set
        + [
            pltpu.VMEM((1008,), jnp.float32),
            pltpu.VMEM_SHARED((NPAD, F), jnp.float32),
            pltpu.VMEM_SHARED((N,), jnp.float32),
        ]
    ),
)

_prep = pl.pallas_call(
    _prep_body,
    out_shape=[
        jax.ShapeDtypeStruct((N, F), jnp.float32),
        jax.ShapeDtypeStruct((N, 1), jnp.float32),
        jax.ShapeDtypeStruct((N, 1), jnp.float32),
    ],
)

_fin = pl.pallas_call(
    _fin_body,
    out_shape=jax.ShapeDtypeStruct((N, F), jnp.float32),
)


def kernel(model_input, edge_index, W, a):
    a1 = a[0, :F].reshape(F, 1)
    a2 = a[0, F:].reshape(F, 1)
    h, s1, s2 = _prep(model_input, W, a1, a2)
    acc, rs = _sc_edges(h, s1.reshape(N), s2.reshape(N), edge_index)
    return _fin(acc, rs.reshape(2, N, 1))


# speedup vs baseline: 17.9631x; 1.0005x over previous
import jax
import jax.numpy as jnp
from jax import lax
from jax.experimental import pallas as pl
from jax.experimental.pallas import tpu as pltpu
from jax.experimental.pallas import tpu_sc as plsc

N = 10000
E = 320000
F = 128
ALPHA = 0.2
C = 128
NCHUNKS = E // C
NW = 32
BASE_CHUNKS = NCHUNKS // NW
EXTRA = NCHUNKS - BASE_CHUNKS * NW
CP_ROWS = 1000
NPAD = 10240


def _prep_body(x_ref, w_ref, a1_ref, a2_ref, h_ref, s1_ref, s2_ref):
    h = jnp.dot(x_ref[...], w_ref[...], preferred_element_type=jnp.float32)
    h_ref[...] = h
    s1_ref[...] = jnp.dot(h, a1_ref[...], preferred_element_type=jnp.float32)
    s2_ref[...] = jnp.dot(h, a2_ref[...], preferred_element_type=jnp.float32)


def _fin_body(acc_ref, rs_ref, o_ref):
    acc = acc_ref[0][:N] + acc_ref[1][:N]
    rs = rs_ref[0] + rs_ref[1]
    o_ref[...] = jnp.maximum(acc / rs, 0.0)


def _splat(vec, i):
    return lax.gather(
        vec, jnp.full((16, 1), i, jnp.int32),
        lax.GatherDimensionNumbers(offset_dims=(),
                                   collapsed_slice_dims=(0,),
                                   start_index_map=(0,)),
        (1,), mode=lax.GatherScatterMode.PROMISE_IN_BOUNDS)


def _sc_body(h_hbm, s1_hbm, s2_hbm, ei_hbm, acc_out, rs_out, *bufs):
    (s1A, s2A, eeA, rbfA, g1A, g2A, g3A, r1A, r2A,
     s1B, s2B, eeB, rbfB, g1B, g2B, g3B, r1B, r2B,
     ix0, gi0, ix1, gi1, ix2, gi2,
     zbuf1, acc_sh, rs_sh) = bufs
    A = (s1A, s2A, eeA, rbfA, g1A, g2A, g3A, r1A, r2A)
    B = (s1B, s2B, eeB, rbfB, g1B, g2B, g3B, r1B, r2B)
    I = ((ix0, gi0), (ix1, gi1), (ix2, gi2))
    c = lax.axis_index("c")
    s = lax.axis_index("s")
    wid = s * 2 + c

    zv = jnp.zeros((16,), jnp.float32)

    def zrow(r, carry):
        for k in range(8):
            rbfA[r, pl.ds(k * 16, 16)] = zv
        return carry

    lax.fori_loop(0, C, zrow, 0)

    def zrow1(i, carry):
        zbuf1[pl.ds(i * 16, 16)] = zv
        return carry

    lax.fori_loop(0, 63, zrow1, 0)

    for j in range(5):
        pltpu.async_copy(rbfA, acc_sh.at[pl.ds(s * 640 + j * C, C)], g1A)
    for j in range(5):
        pltpu.make_async_copy(rbfA, acc_sh.at[pl.ds(s * 640 + j * C, C)], g1A).wait()

    @pl.when(s < 10)
    def _zero_spmem():
        pltpu.sync_copy(zbuf1.at[pl.ds(0, CP_ROWS)], rs_sh.at[pl.ds(s * CP_ROWS, CP_ROWS)])

    plsc.subcore_barrier()

    base = wid * BASE_CHUNKS

    def issue_idx(cid, iset):
        idx2, gi = iset
        pltpu.async_copy(ei_hbm.at[:, pl.ds(cid * C, C)], idx2, gi)

    def wait_idx(cid, iset):
        idx2, gi = iset
        pltpu.make_async_copy(ei_hbm.at[:, pl.ds(cid * C, C)], idx2, gi).wait()

    def issue_gathers(ds, iset):
        s1v, s2v, eev, rows_v, g1, g2, g3, r1, r2 = ds
        idx2, gi = iset
        pltpu.async_copy(s1_hbm.at[idx2.at[0]], s1v, g1)
        pltpu.async_copy(s2_hbm.at[idx2.at[1]], s2v, g2)
        pltpu.async_copy(h_hbm.at[idx2.at[1]], rows_v, g3)

    def process(ds, iset):
        s1v, s2v, eev, rows_v, g1, g2, g3, r1, r2 = ds
        idx2, gi = iset
        pltpu.make_async_copy(s1_hbm.at[idx2.at[0]], s1v, g1).wait()
        pltpu.make_async_copy(s2_hbm.at[idx2.at[1]], s2v, g2).wait()
        for g in range(C // 16):
            sl = pl.ds(g * 16, 16)
            x = s1v[sl] + s2v[sl]
            eev[sl] = jnp.exp(-jnp.maximum(x, ALPHA * x))
        pltpu.make_async_copy(h_hbm.at[idx2.at[1]], rows_v, g3).wait()

        def scale_body(g, carry2):
            eg = eev[pl.ds(g * 16, 16)]
            for i in range(16):
                sv = _splat(eg, i)
                e = g * 16 + i
                for cc in range(F // 16):
                    sl2 = pl.ds(cc * 16, 16)
                    rows_v[e, sl2] = rows_v[e, sl2] * sv
            return carry2

        lax.fori_loop(0, C // 16, scale_body, 0)
        pltpu.async_copy(eev, rs_sh.at[idx2.at[0]], r1, add=True)
        pltpu.async_copy(rows_v, acc_sh.at[idx2.at[0]], r2, add=True)

    def wait_scatter(ds, iset):
        s1v, s2v, eev, rows_v, g1, g2, g3, r1, r2 = ds
        idx2, gi = iset
        pltpu.make_async_copy(eev, rs_sh.at[idx2.at[0]], r1).wait()
        pltpu.make_async_copy(rows_v, acc_sh.at[idx2.at[0]], r2).wait()

    issue_idx(base, I[0])
    issue_idx(base + 1, I[1])
    wait_idx(base, I[0])
    issue_gathers(A, I[0])

    def six_body(k, carry):
        j0 = base + 6 * k
        for jj in range(6):
            d_cur = (A, B)[jj % 2]
            d_nxt = (A, B)[(jj + 1) % 2]
            i_cur = I[jj % 3]
            i_nxt = I[(jj + 1) % 3]
            i_nx2 = I[(jj + 2) % 3]

            if jj > 0:
                wait_scatter(d_nxt, i_nx2)
            else:
                @pl.when(k > 0)
                def _drain_prev():
                    wait_scatter(d_nxt, i_nx2)

            wait_idx(j0 + jj + 1, i_nxt)
            issue_gathers(d_nxt, i_nxt)
            issue_idx(j0 + jj + 2, i_nx2)
            process(d_cur, i_cur)
        return carry

    lax.fori_loop(0, BASE_CHUNKS // 6, six_body, 0)

    wait_scatter(B, I[2])
    pltpu.make_async_copy(s1_hbm.at[ix0.at[0]], s1A, g1A).wait()
    pltpu.make_async_copy(s2_hbm.at[ix0.at[1]], s2A, g2A).wait()
    pltpu.make_async_copy(h_hbm.at[ix0.at[1]], rbfA, g3A).wait()
    wait_idx(base + BASE_CHUNKS + 1, I[1])

    @pl.when(wid < EXTRA)
    def _tail():
        cid = NW * BASE_CHUNKS + wid
        issue_idx(cid, I[0])
        wait_idx(cid, I[0])
        issue_gathers(A, I[0])
        process(A, I[0])
        wait_scatter(A, I[0])

    plsc.subcore_barrier()

    stages = ((rbfA, g1A), (rbfB, g1B))
    for j in range(5):
        stage, sem = stages[j % 2]
        if j >= 2:
            pltpu.make_async_copy(
                stage, acc_out.at[c, pl.ds(s * 640 + (j - 2) * C, C)], sem).wait()
        pltpu.sync_copy(acc_sh.at[pl.ds(s * 640 + j * C, C)], stage)
        pltpu.async_copy(stage, acc_out.at[c, pl.ds(s * 640 + j * C, C)], sem)
    for j in (3, 4):
        stage, sem = stages[j % 2]
        pltpu.make_async_copy(
            stage, acc_out.at[c, pl.ds(s * 640 + j * C, C)], sem).wait()

    @pl.when(s < 10)
    def _copy_out_rs():
        pltpu.sync_copy(rs_sh.at[pl.ds(s * CP_ROWS, CP_ROWS)],
                        zbuf1.at[pl.ds(0, CP_ROWS)])
        pltpu.sync_copy(zbuf1.at[pl.ds(0, CP_ROWS)],
                        rs_out.at[pl.ds(c * N + s * CP_ROWS, CP_ROWS)])


_data_set = [
    pltpu.VMEM((C,), jnp.float32),
    pltpu.VMEM((C,), jnp.float32),
    pltpu.VMEM((C,), jnp.float32),
    pltpu.VMEM((C, F), jnp.float32),
    pltpu.SemaphoreType.DMA,
    pltpu.SemaphoreType.DMA,
    pltpu.SemaphoreType.DMA,
    pltpu.SemaphoreType.DMA,
    pltpu.SemaphoreType.DMA,
]
_idx_set = [
    pltpu.VMEM((2, C), jnp.int32),
    pltpu.SemaphoreType.DMA,
]

_sc_edges = pl.kernel(
    _sc_body,
    out_type=[
        jax.ShapeDtypeStruct((2, NPAD, F), jnp.float32),
        jax.ShapeDtypeStruct((2 * N,), jnp.float32),
    ],
    mesh=plsc.VectorSubcoreMesh(core_axis_name="c", subcore_axis_name="s"),
    scratch_types=(
        2 * _data_set
        + 3 * _idx_set
        + [
            pltpu.VMEM((1008,), jnp.float32),
            pltpu.VMEM_SHARED((NPAD, F), jnp.float32),
            pltpu.VMEM_SHARED((N,), jnp.float32),
        ]
    ),
)

_prep = pl.pallas_call(
    _prep_body,
    out_shape=[
        jax.ShapeDtypeStruct((N, F), jnp.float32),
        jax.ShapeDtypeStruct((N, 1), jnp.float32),
        jax.ShapeDtypeStruct((N, 1), jnp.float32),
    ],
)

_fin = pl.pallas_call(
    _fin_body,
    out_shape=jax.ShapeDtypeStruct((N, F), jnp.float32),
)


def kernel(model_input, edge_index, W, a):
    a1 = a[0, :F].reshape(F, 1)
    a2 = a[0, F:].reshape(F, 1)
    h, s1, s2 = _prep(model_input, W, a1, a2)
    acc, rs = _sc_edges(h, s1.reshape(N), s2.reshape(N), edge_index)
    return _fin(acc, rs.reshape(2, N, 1))


# speedup vs baseline: 18.1143x; 1.0084x over previous
import jax
import jax.numpy as jnp
from jax import lax
from jax.experimental import pallas as pl
from jax.experimental.pallas import tpu as pltpu
from jax.experimental.pallas import tpu_sc as plsc

N = 10000
E = 320000
F = 128
ALPHA = 0.2
C = 128
NCHUNKS = E // C
NW = 32
BASE_CHUNKS = NCHUNKS // NW
EXTRA = NCHUNKS - BASE_CHUNKS * NW
CP_ROWS = 1000
NPAD = 10240


def _prep_body(x_ref, w_ref, a_ref, h_ref, s1_ref, s2_ref):
    h = jnp.dot(x_ref[...], w_ref[...], preferred_element_type=jnp.float32)
    h_ref[...] = h
    a2f = a_ref[...].reshape(2, F).T
    s12 = jnp.dot(h, a2f, preferred_element_type=jnp.float32)
    s1_ref[...] = s12[:, :1]
    s2_ref[...] = s12[:, 1:]


def _fin_body(acc_ref, rs_ref, o_ref):
    acc = acc_ref[0][:N] + acc_ref[1][:N]
    rs = rs_ref[0] + rs_ref[1]
    o_ref[...] = jnp.maximum(acc / rs, 0.0)


def _splat(vec, i):
    return lax.gather(
        vec, jnp.full((16, 1), i, jnp.int32),
        lax.GatherDimensionNumbers(offset_dims=(),
                                   collapsed_slice_dims=(0,),
                                   start_index_map=(0,)),
        (1,), mode=lax.GatherScatterMode.PROMISE_IN_BOUNDS)


def _sc_body(h_hbm, s1_hbm, s2_hbm, ei_hbm, acc_out, rs_out, *bufs):
    (s1A, s2A, eeA, rbfA, g1A, g2A, g3A, r1A, r2A,
     s1B, s2B, eeB, rbfB, g1B, g2B, g3B, r1B, r2B,
     ix0, gi0, ix1, gi1, ix2, gi2,
     zbuf1, acc_sh, rs_sh) = bufs
    A = (s1A, s2A, eeA, rbfA, g1A, g2A, g3A, r1A, r2A)
    B = (s1B, s2B, eeB, rbfB, g1B, g2B, g3B, r1B, r2B)
    I = ((ix0, gi0), (ix1, gi1), (ix2, gi2))
    c = lax.axis_index("c")
    s = lax.axis_index("s")
    wid = s * 2 + c

    base = wid * BASE_CHUNKS

    def issue_idx(cid, iset):
        idx2, gi = iset
        pltpu.async_copy(ei_hbm.at[:, pl.ds(cid * C, C)], idx2, gi)

    def wait_idx(cid, iset):
        idx2, gi = iset
        pltpu.make_async_copy(ei_hbm.at[:, pl.ds(cid * C, C)], idx2, gi).wait()

    def issue_gathers(ds, iset):
        s1v, s2v, eev, rows_v, g1, g2, g3, r1, r2 = ds
        idx2, gi = iset
        pltpu.async_copy(s1_hbm.at[idx2.at[0]], s1v, g1)
        pltpu.async_copy(s2_hbm.at[idx2.at[1]], s2v, g2)
        pltpu.async_copy(h_hbm.at[idx2.at[1]], rows_v, g3)

    def process(ds, iset):
        s1v, s2v, eev, rows_v, g1, g2, g3, r1, r2 = ds
        idx2, gi = iset
        pltpu.make_async_copy(s1_hbm.at[idx2.at[0]], s1v, g1).wait()
        pltpu.make_async_copy(s2_hbm.at[idx2.at[1]], s2v, g2).wait()
        for g in range(C // 16):
            sl = pl.ds(g * 16, 16)
            x = s1v[sl] + s2v[sl]
            eev[sl] = jnp.exp(-jnp.maximum(x, ALPHA * x))
        pltpu.make_async_copy(h_hbm.at[idx2.at[1]], rows_v, g3).wait()

        def scale_body(g, carry2):
            eg = eev[pl.ds(g * 16, 16)]
            for i in range(16):
                sv = _splat(eg, i)
                e = g * 16 + i
                for cc in range(F // 16):
                    sl2 = pl.ds(cc * 16, 16)
                    rows_v[e, sl2] = rows_v[e, sl2] * sv
            return carry2

        lax.fori_loop(0, C // 16, scale_body, 0)
        pltpu.async_copy(eev, rs_sh.at[idx2.at[0]], r1, add=True)
        pltpu.async_copy(rows_v, acc_sh.at[idx2.at[0]], r2, add=True)

    def wait_scatter(ds, iset):
        s1v, s2v, eev, rows_v, g1, g2, g3, r1, r2 = ds
        idx2, gi = iset
        pltpu.make_async_copy(eev, rs_sh.at[idx2.at[0]], r1).wait()
        pltpu.make_async_copy(rows_v, acc_sh.at[idx2.at[0]], r2).wait()

    issue_idx(base, I[0])
    issue_idx(base + 1, I[1])

    zv = jnp.zeros((16,), jnp.float32)

    def zrow(r, carry):
        for k in range(8):
            rbfA[r, pl.ds(k * 16, 16)] = zv
        return carry

    lax.fori_loop(0, C, zrow, 0)

    def zrow1(i, carry):
        zbuf1[pl.ds(i * 16, 16)] = zv
        return carry

    lax.fori_loop(0, 63, zrow1, 0)

    for j in range(5):
        pltpu.async_copy(rbfA, acc_sh.at[pl.ds(s * 640 + j * C, C)], g1A)
    for j in range(5):
        pltpu.make_async_copy(rbfA, acc_sh.at[pl.ds(s * 640 + j * C, C)], g1A).wait()

    @pl.when(s < 10)
    def _zero_spmem():
        pltpu.sync_copy(zbuf1.at[pl.ds(0, CP_ROWS)], rs_sh.at[pl.ds(s * CP_ROWS, CP_ROWS)])

    plsc.subcore_barrier()

    wait_idx(base, I[0])
    issue_gathers(A, I[0])

    def six_body(k, carry):
        j0 = base + 6 * k
        for jj in range(6):
            d_cur = (A, B)[jj % 2]
            d_nxt = (A, B)[(jj + 1) % 2]
            i_cur = I[jj % 3]
            i_nxt = I[(jj + 1) % 3]
            i_nx2 = I[(jj + 2) % 3]

            if jj > 0:
                wait_scatter(d_nxt, i_nx2)
            else:
                @pl.when(k > 0)
                def _drain_prev():
                    wait_scatter(d_nxt, i_nx2)

            wait_idx(j0 + jj + 1, i_nxt)
            issue_gathers(d_nxt, i_nxt)
            issue_idx(j0 + jj + 2, i_nx2)
            process(d_cur, i_cur)
        return carry

    lax.fori_loop(0, BASE_CHUNKS // 6, six_body, 0)

    wait_scatter(B, I[2])
    pltpu.make_async_copy(s1_hbm.at[ix0.at[0]], s1A, g1A).wait()
    pltpu.make_async_copy(s2_hbm.at[ix0.at[1]], s2A, g2A).wait()
    pltpu.make_async_copy(h_hbm.at[ix0.at[1]], rbfA, g3A).wait()
    wait_idx(base + BASE_CHUNKS + 1, I[1])

    @pl.when(wid < EXTRA)
    def _tail():
        cid = NW * BASE_CHUNKS + wid
        issue_idx(cid, I[0])
        wait_idx(cid, I[0])
        issue_gathers(A, I[0])
        process(A, I[0])
        wait_scatter(A, I[0])

    plsc.subcore_barrier()

    stages = ((rbfA, g1A), (rbfB, g1B))
    for j in range(5):
        stage, sem = stages[j % 2]
        if j >= 2:
            pltpu.make_async_copy(
                stage, acc_out.at[c, pl.ds(s * 640 + (j - 2) * C, C)], sem).wait()
        pltpu.sync_copy(acc_sh.at[pl.ds(s * 640 + j * C, C)], stage)
        pltpu.async_copy(stage, acc_out.at[c, pl.ds(s * 640 + j * C, C)], sem)
    for j in (3, 4):
        stage, sem = stages[j % 2]
        pltpu.make_async_copy(
            stage, acc_out.at[c, pl.ds(s * 640 + j * C, C)], sem).wait()

    @pl.when(s < 10)
    def _copy_out_rs():
        pltpu.sync_copy(rs_sh.at[pl.ds(s * CP_ROWS, CP_ROWS)],
                        zbuf1.at[pl.ds(0, CP_ROWS)])
        pltpu.sync_copy(zbuf1.at[pl.ds(0, CP_ROWS)],
                        rs_out.at[pl.ds(c * N + s * CP_ROWS, CP_ROWS)])


_data_set = [
    pltpu.VMEM((C,), jnp.float32),
    pltpu.VMEM((C,), jnp.float32),
    pltpu.VMEM((C,), jnp.float32),
    pltpu.VMEM((C, F), jnp.float32),
    pltpu.SemaphoreType.DMA,
    pltpu.SemaphoreType.DMA,
    pltpu.SemaphoreType.DMA,
    pltpu.SemaphoreType.DMA,
    pltpu.SemaphoreType.DMA,
]
_idx_set = [
    pltpu.VMEM((2, C), jnp.int32),
    pltpu.SemaphoreType.DMA,
]

_sc_edges = pl.kernel(
    _sc_body,
    out_type=[
        jax.ShapeDtypeStruct((2, NPAD, F), jnp.float32),
        jax.ShapeDtypeStruct((2 * N,), jnp.float32),
    ],
    mesh=plsc.VectorSubcoreMesh(core_axis_name="c", subcore_axis_name="s"),
    scratch_types=(
        2 * _data_set
        + 3 * _idx_set
        + [
            pltpu.VMEM((1008,), jnp.float32),
            pltpu.VMEM_SHARED((NPAD, F), jnp.float32),
            pltpu.VMEM_SHARED((N,), jnp.float32),
        ]
    ),
)

_prep = pl.pallas_call(
    _prep_body,
    out_shape=[
        jax.ShapeDtypeStruct((N, F), jnp.float32),
        jax.ShapeDtypeStruct((N, 1), jnp.float32),
        jax.ShapeDtypeStruct((N, 1), jnp.float32),
    ],
)

_fin = pl.pallas_call(
    _fin_body,
    out_shape=jax.ShapeDtypeStruct((N, F), jnp.float32),
)


def kernel(model_input, edge_index, W, a):
    h, s1, s2 = _prep(model_input, W, a)
    acc, rs = _sc_edges(h, s1.reshape(N), s2.reshape(N), edge_index)
    return _fin(acc, rs.reshape(2, N, 1))
